# baseline (speedup 1.0000x reference)
import functools

import jax
import jax.numpy as jnp
from jax import lax
from jax.experimental import pallas as pl
from jax.experimental.pallas import tpu as pltpu
from jax.experimental.pallas import tpu_sc as plsc

GAMMA_CONST = 12.0
EPS = 1e-12


def _normalize_body(x_ref, o_ref):
    v = x_ref[...]
    n = jnp.sqrt(jnp.sum(v * v, axis=1, keepdims=True))
    v = v / jnp.maximum(n, EPS)
    nrows, d = v.shape
    blocks = []
    for m in range(d // 32):
        a = v[:, 32 * m:32 * m + 16]
        b = v[:, 32 * m + 16:32 * m + 32]
        blocks.append(jnp.stack([a, b], axis=-1).reshape(nrows, 32))
    o_ref[...] = jnp.concatenate(blocks, axis=1).astype(jnp.bfloat16)


def _normalize_rows(x):
    n_nodes, d = x.shape
    blk = 1000
    return pl.pallas_call(
        _normalize_body,
        grid=(n_nodes // blk,),
        in_specs=[pl.BlockSpec((blk, d), lambda i: (i, 0))],
        out_specs=pl.BlockSpec((blk, d), lambda i: (i, 0)),
        out_shape=jax.ShapeDtypeStruct((n_nodes, d), jnp.bfloat16),
    )(x)


def _make_sc_scorer(n_nodes, d, e_total):
    info = plsc.get_sparse_core_info()
    nc, ns, lanes = info.num_cores, info.num_subcores, info.num_lanes
    nw = nc * ns
    assert e_total % nw == 0
    epw = e_total // nw
    ch = 64
    tail_ch = epw - (epw // ch) * ch
    n_full = epw // ch
    assert n_full % 2 == 0 and tail_ch % lanes == 0 and tail_ch > 0
    hw = d // 2

    mesh = plsc.VectorSubcoreMesh(core_axis_name="c", subcore_axis_name="s")

    buf_types = [
        pltpu.VMEM((ch, d), jnp.int32),
        pltpu.VMEM((ch, d), jnp.int32),
        pltpu.VMEM((ch, 2 * d), jnp.float32),
        pltpu.VMEM((ch,), jnp.int32),
        pltpu.VMEM((ch,), jnp.int32),
        pltpu.VMEM((ch,), jnp.float32),
        pltpu.SemaphoreType.DMA,
        pltpu.SemaphoreType.DMA,
        pltpu.SemaphoreType.DMA,
        pltpu.SemaphoreType.DMA,
    ]

    @functools.partial(
        pl.kernel,
        mesh=mesh,
        compiler_params=pltpu.CompilerParams(needs_layout_passes=False),
        out_type=jax.ShapeDtypeStruct((e_total,), jnp.float32),
        scratch_types=[
            pltpu.VMEM((epw,), jnp.int32),
            pltpu.VMEM((epw,), jnp.int32),
            pltpu.VMEM_SHARED((n_nodes // 2, d), jnp.int32),
        ] + buf_types + buf_types,
    )
    def scorer(xp_hbm, src_hbm, dst_hbm, rel_hbm, out_hbm,
               src_v, dst_v, xn_sh,
               head0, tail0, rel0, ri0, ti0, out0, sh0, st0, sr0, so0,
               head1, tail1, rel1, ri1, ti1, out1, sh1, st1, sr1, so1):
        sid = lax.axis_index("s")
        wid = sid * nc + lax.axis_index("c")
        base = wid * epw

        @pl.when(sid == 0)
        def _():
            pltpu.sync_copy(xp_hbm, xn_sh)

        pltpu.sync_copy(src_hbm.at[pl.ds(base, epw)], src_v)
        pltpu.sync_copy(dst_hbm.at[pl.ds(base, epw)], dst_v)
        plsc.subcore_barrier()
        bufs = ((head0, tail0, rel0, ri0, ti0, out0, sh0, st0, sr0, so0),
                (head1, tail1, rel1, ri1, ti1, out1, sh1, st1, sr1, so1))

        def copies(off, buf, n=ch):
            head_b, tail_b, rel_b, ri_b, ti_b, sh, st, sr = (
                buf[0], buf[1], buf[2], buf[3], buf[4],
                buf[6], buf[7], buf[8])
            return (
                pltpu.make_async_copy(
                    xn_sh.at[ri_b.at[pl.ds(0, n)]],
                    head_b.at[pl.ds(0, n)], sh),
                pltpu.make_async_copy(
                    xn_sh.at[ti_b.at[pl.ds(0, n)]],
                    tail_b.at[pl.ds(0, n)], st),
                pltpu.make_async_copy(
                    rel_hbm.at[pl.ds(base + off, n)],
                    rel_b.at[pl.ds(0, n)], sr),
            )

        def out_copy(off, buf, n=ch):
            return pltpu.make_async_copy(
                buf[5].at[pl.ds(0, n)],
                out_hbm.at[pl.ds(base + off, n)], buf[9])

        def issue(off, buf, n=ch):
            ri_b, ti_b = buf[3], buf[4]
            for q in range(n // lanes):
                s16 = src_v[pl.ds(off + q * lanes, lanes)]
                d16 = dst_v[pl.ds(off + q * lanes, lanes)]
                ri_b[pl.ds(q * lanes, lanes)] = jnp.right_shift(s16, 1)
                ti_b[pl.ds(q * lanes, lanes)] = jnp.right_shift(d16, 1)
            for cp in copies(off, buf, n):
                cp.start()

        def compute(off, buf, n=ch):
            head_b, tail_b, rel_b, out_b = buf[0], buf[1], buf[2], buf[5]
            lane_ids = lax.iota(jnp.int32, 16)

            def group_body(g, _):
                srcs = src_v[pl.ds(off + g * lanes, lanes)]
                dsts = dst_v[pl.ds(off + g * lanes, lanes)]
                res = jnp.zeros((lanes,), jnp.float32)
                for e16 in range(lanes):
                    e = g * lanes + e16
                    sp = (srcs[e16] & 1) * hw
                    dp = (dsts[e16] & 1) * hw
                    acc = jnp.zeros((lanes,), jnp.float32)
                    for m in range(d // 32):
                        hp = plsc.bitcast(
                            head_b[e, pl.ds(sp + lanes * m, lanes)],
                            jnp.bfloat16)
                        tp = plsc.bitcast(
                            tail_b[e, pl.ds(dp + lanes * m, lanes)],
                            jnp.bfloat16)
                        ha, hb = plsc.unpack(
                            hp, format=plsc.PackFormat.INTERLEAVED)
                        ta, tb = plsc.unpack(
                            tp, format=plsc.PackFormat.INTERLEAVED)
                        ra0 = rel_b[e, pl.ds(32 * m, lanes)]
                        ra1 = rel_b[e, pl.ds(32 * m + lanes, lanes)]
                        rb0 = rel_b[e, pl.ds(d + 32 * m, lanes)]
                        rb1 = rel_b[e, pl.ds(d + 32 * m + lanes, lanes)]
                        acc = acc + jnp.abs(ha * ra0 - ta * rb0)
                        acc = acc + jnp.abs(hb * ra1 - tb * rb1)
                    s = GAMMA_CONST - jnp.sum(acc)
                    res = jnp.where(lane_ids == e16, s, res)
                out_b[pl.ds(g * lanes, lanes)] = res
                return 0

            lax.fori_loop(0, n // lanes, group_body, 0)

        def drain(off, buf, n=ch):
            for cp in copies(off, buf, n):
                cp.wait()

        issue(0, bufs[0])

        def pair_body(i, _):
            off0 = (2 * i) * ch
            issue(off0 + ch, bufs[1])
            drain(off0, bufs[0])

            @pl.when(i > 0)
            def _():
                out_copy(off0, bufs[0]).wait()

            compute(off0, bufs[0])
            out_copy(off0, bufs[0]).start()

            @pl.when(i < n_full // 2 - 1)
            def _():
                issue(off0 + 2 * ch, bufs[0])

            drain(off0 + ch, bufs[1])

            @pl.when(i > 0)
            def _():
                out_copy(off0 + ch, bufs[1]).wait()

            compute(off0 + ch, bufs[1])
            out_copy(off0 + ch, bufs[1]).start()
            return 0

        lax.fori_loop(0, n_full // 2, pair_body, 0)
        tail_off = n_full * ch
        out_copy(tail_off, bufs[0]).wait()
        issue(tail_off, bufs[0], tail_ch)
        drain(tail_off, bufs[0], tail_ch)
        compute(tail_off, bufs[0], tail_ch)
        out_copy(tail_off, bufs[0], tail_ch).start()
        out_copy(tail_off, bufs[0], tail_ch).wait()
        out_copy(tail_off - ch, bufs[1]).wait()

    return scorer


def kernel(x, edge_index, edge_attr):
    n_nodes, d = x.shape
    e_total = edge_attr.shape[0]
    xn = _normalize_rows(x)
    xp = lax.bitcast_convert_type(
        xn.reshape(n_nodes // 2, d, 2), jnp.int32)
    src = edge_index[0].astype(jnp.int32)
    dst = edge_index[1].astype(jnp.int32)
    scorer = _make_sc_scorer(n_nodes, d, e_total)
    score = scorer(xp, src, dst, edge_attr)
    return score.reshape(e_total, 1)


# speedup vs baseline: 3.4146x; 3.4146x over previous
import functools

import jax
import jax.numpy as jnp
from jax import lax
from jax.experimental import pallas as pl
from jax.experimental.pallas import tpu as pltpu
from jax.experimental.pallas import tpu_sc as plsc

GAMMA_CONST = 12.0
EPS = 1e-12


def _normalize_body(x_ref, o_ref):
    v = x_ref[...]
    n = jnp.sqrt(jnp.sum(v * v, axis=1, keepdims=True))
    o_ref[...] = v / jnp.maximum(n, EPS)


def _normalize_rows(x):
    n_nodes, d = x.shape
    return pl.pallas_call(
        _normalize_body,
        out_shape=jax.ShapeDtypeStruct((n_nodes, d), jnp.float32),
    )(x)


def _make_sc_scorer(n_nodes, d, e_total):
    info = plsc.get_sparse_core_info()
    nc, ns, lanes = info.num_cores, info.num_subcores, info.num_lanes
    nw = nc * ns
    assert e_total % nw == 0
    epw = e_total // nw
    ch = 80
    assert epw % ch == 0
    n_chunks = epw // ch
    n_seg = d // lanes

    mesh = plsc.VectorSubcoreMesh(core_axis_name="c", subcore_axis_name="s")

    buf_types = [
        pltpu.VMEM((ch, d), jnp.float32),
        pltpu.VMEM((ch, d), jnp.float32),
        pltpu.VMEM((ch, 2 * d), jnp.float32),
        pltpu.SemaphoreType.DMA,
        pltpu.SemaphoreType.DMA,
        pltpu.SemaphoreType.DMA,
    ]

    @functools.partial(
        pl.kernel,
        mesh=mesh,
        compiler_params=pltpu.CompilerParams(needs_layout_passes=False),
        out_type=jax.ShapeDtypeStruct((e_total,), jnp.float32),
        scratch_types=[
            pltpu.VMEM((epw,), jnp.int32),
            pltpu.VMEM((epw,), jnp.int32),
            pltpu.VMEM((epw,), jnp.float32),
        ] + buf_types + buf_types,
    )
    def scorer(xn_hbm, src_hbm, dst_hbm, rel_hbm, out_hbm,
               src_v, dst_v, out_v,
               head0, tail0, rel0, sh0, st0, sr0,
               head1, tail1, rel1, sh1, st1, sr1):
        wid = lax.axis_index("s") * nc + lax.axis_index("c")
        base = wid * epw
        pltpu.sync_copy(src_hbm.at[pl.ds(base, epw)], src_v)
        pltpu.sync_copy(dst_hbm.at[pl.ds(base, epw)], dst_v)
        bufs = ((head0, tail0, rel0, sh0, st0, sr0),
                (head1, tail1, rel1, sh1, st1, sr1))

        def copies(off, buf):
            head_b, tail_b, rel_b, sh, st, sr = buf
            return (
                pltpu.make_async_copy(
                    xn_hbm.at[src_v.at[pl.ds(off, ch)]], head_b, sh),
                pltpu.make_async_copy(
                    xn_hbm.at[dst_v.at[pl.ds(off, ch)]], tail_b, st),
                pltpu.make_async_copy(
                    rel_hbm.at[pl.ds(base + off, ch)], rel_b, sr),
            )

        def issue(off, buf):
            for cp in copies(off, buf):
                cp.start()

        def compute(off, buf):
            head_b, tail_b, rel_b, _, _, _ = buf
            lane_ids = lax.iota(jnp.int32, 16)

            def group_body(g, _):
                def edge_body(e16, res):
                    e = g * lanes + e16
                    acc = jnp.zeros((lanes,), jnp.float32)
                    for j in range(n_seg):
                        h = head_b[e, pl.ds(j * lanes, lanes)]
                        t = tail_b[e, pl.ds(j * lanes, lanes)]
                        ra = rel_b[e, pl.ds(j * lanes, lanes)]
                        rb = rel_b[e, pl.ds(d + j * lanes, lanes)]
                        acc = acc + jnp.abs(h * ra - t * rb)
                    s = GAMMA_CONST - jnp.sum(acc)
                    return jnp.where(lane_ids == e16, s, res)

                res = lax.fori_loop(0, lanes, edge_body,
                                    jnp.zeros((lanes,), jnp.float32))
                out_v[pl.ds(off + g * lanes, lanes)] = res
                return 0

            lax.fori_loop(0, ch // lanes, group_body, 0)

        def drain(off, buf):
            for cp in copies(off, buf):
                cp.wait()

        issue(0, bufs[0])

        def pair_body(i, _):
            off0 = (2 * i) * ch
            issue(off0 + ch, bufs[1])
            drain(off0, bufs[0])
            compute(off0, bufs[0])
            issue(off0 + 2 * ch, bufs[0])
            drain(off0 + ch, bufs[1])
            compute(off0 + ch, bufs[1])
            return 0

        lax.fori_loop(0, (n_chunks - 1) // 2, pair_body, 0)
        last = (n_chunks - 1) * ch
        drain(last, bufs[0])
        compute(last, bufs[0])
        pltpu.sync_copy(out_v, out_hbm.at[pl.ds(base, epw)])

    return scorer


def kernel(x, edge_index, edge_attr):
    n_nodes, d = x.shape
    e_total = edge_attr.shape[0]
    xn = _normalize_rows(x)
    src = edge_index[0].astype(jnp.int32)
    dst = edge_index[1].astype(jnp.int32)
    scorer = _make_sc_scorer(n_nodes, d, e_total)
    score = scorer(xn, src, dst, edge_attr)
    return score.reshape(e_total, 1)
